# baseline (speedup 1.0000x reference)
import functools
import math

import jax
import jax.numpy as jnp
from jax import lax
from jax.experimental import pallas as pl
from jax.experimental.pallas import tpu as pltpu
from jax.experimental.pallas import tpu_sc as plsc

_EMBED = 64
_NW = 32
_CHUNK = 256


def _make_embed(total):
    b_per_w = total // _NW
    n_chunks = b_per_w // _CHUNK
    mesh = plsc.VectorSubcoreMesh(core_axis_name="c", subcore_axis_name="s")

    @functools.partial(
        pl.kernel,
        mesh=mesh,
        out_type=jax.ShapeDtypeStruct((total // 2, 128), jnp.float32),
        scratch_types=[
            pltpu.VMEM((_CHUNK,), jnp.int32),
            pltpu.VMEM((_CHUNK, 128), jnp.float32),
            pltpu.VMEM((_CHUNK // 2, 128), jnp.float32),
            pltpu.SemaphoreType.DMA,
        ],
    )
    def embed(idx_hbm, table_hbm, out_hbm, idx_v, g_v, o_v, sem):
        wid = lax.axis_index("s") * 2 + lax.axis_index("c")
        base = wid * b_per_w

        def chunk_body(c, carry):
            off = pl.multiple_of(base + c * _CHUNK, _CHUNK)
            off2 = pl.multiple_of(base // 2 + c * (_CHUNK // 2), _CHUNK // 2)
            pltpu.sync_copy(idx_hbm.at[pl.ds(off, _CHUNK)], idx_v)
            hi = idx_v[...]
            pltpu.async_copy(table_hbm.at[idx_v], g_v, sem).wait()
            pltpu.sync_copy(g_v.at[pl.ds(0, _CHUNK // 2)], out_hbm.at[pl.ds(off2, _CHUNK // 2)])
            return carry

        lax.fori_loop(0, n_chunks, chunk_body, 0)

    return embed


def kernel(input_token, table):
    batch, seq = input_token.shape
    total = batch * seq
    idx = input_token.reshape(total).astype(jnp.int32)
    table2 = table.reshape(table.shape[0] // 2, 128)
    out = _make_embed(total)(idx, table2)
    return out.reshape(batch, seq, _EMBED)


# speedup vs baseline: 1.1597x; 1.1597x over previous
import functools
import math

import jax
import jax.numpy as jnp
from jax import lax
from jax.experimental import pallas as pl
from jax.experimental.pallas import tpu as pltpu
from jax.experimental.pallas import tpu_sc as plsc

_EMBED = 64
_SCALE = math.sqrt(_EMBED)
_LANES = 16
_NC = 2
_NS = 16
_NW = _NC * _NS

_CHUNK = 256
_NBUF = 4


def _make_embed(total):
    assert total % (_NW * _CHUNK) == 0
    b_per_w = total // _NW
    n_chunks = b_per_w // _CHUNK
    assert n_chunks % _NBUF == 0 and n_chunks >= 2 * _NBUF
    mesh = plsc.VectorSubcoreMesh(core_axis_name="c", subcore_axis_name="s")

    scratch = (
        [pltpu.VMEM((b_per_w,), jnp.int32)]
        + [pltpu.VMEM((_CHUNK, _EMBED), jnp.float32) for _ in range(_NBUF)]
        + [pltpu.SemaphoreType.DMA for _ in range(2 * _NBUF)]
    )

    @functools.partial(
        pl.kernel,
        mesh=mesh,
        out_type=jax.ShapeDtypeStruct((total, _EMBED), jnp.float32),
        scratch_types=scratch,
        compiler_params=pltpu.CompilerParams(use_tc_tiling_on_sc=False),
    )
    def embed(idx_hbm, table_hbm, out_hbm, idx_v, *bufs_and_sems):
        rows = bufs_and_sems[:_NBUF]
        sem_g = bufs_and_sems[_NBUF:2 * _NBUF]
        sem_s = bufs_and_sems[2 * _NBUF:]
        wid = lax.axis_index("s") * _NC + lax.axis_index("c")
        base = wid * b_per_w

        pltpu.sync_copy(idx_hbm.at[pl.ds(base, b_per_w)], idx_v)

        def gather_start(c, b):
            pltpu.async_copy(
                table_hbm.at[idx_v.at[pl.ds(c * _CHUNK, _CHUNK)]],
                rows[b], sem_g[b])

        def gather_wait(b):
            pltpu.make_async_copy(
                table_hbm.at[idx_v.at[pl.ds(0, _CHUNK)]],
                rows[b], sem_g[b]).wait()

        def scatter_start(c, b):
            pltpu.async_copy(
                rows[b], out_hbm.at[pl.ds(base + c * _CHUNK, _CHUNK)],
                sem_s[b])

        def scatter_wait(b):
            pltpu.make_async_copy(
                rows[b], out_hbm.at[pl.ds(base, _CHUNK)], sem_s[b]).wait()

        def scale(b):
            @plsc.parallel_loop(0, _CHUNK, step=1, unroll=8)
            def _(i):
                for j in range(_EMBED // _LANES):
                    sl = pl.ds(j * _LANES, _LANES)
                    rows[b][i, sl] = rows[b][i, sl] * _SCALE

        gather_start(0, 0)
        gather_start(1, 1)

        def outer(oi, carry):
            c0 = oi * _NBUF
            for j in range(_NBUF):
                c = c0 + j
                b = j
                nb = (j + 2) % _NBUF
                gather_wait(b)

                @pl.when(c >= 2)
                def _():
                    scatter_wait(nb)

                @pl.when(c + 2 < n_chunks)
                def _():
                    gather_start(c + 2, nb)

                scale(b)
                scatter_start(c, b)
            return carry

        lax.fori_loop(0, n_chunks // _NBUF, outer, 0)

        scatter_wait((n_chunks - 2) % _NBUF)
        scatter_wait((n_chunks - 1) % _NBUF)

    return embed


def kernel(input_token, table):
    batch, seq = input_token.shape
    total = batch * seq
    idx = input_token.reshape(total).astype(jnp.int32)
    out = _make_embed(total)(idx, table)
    return out.reshape(batch, seq, _EMBED)
